# baseline (speedup 1.0000x reference)
import functools

import jax
import jax.numpy as jnp
from jax import lax
from jax.experimental import pallas as pl
from jax.experimental.pallas import tpu as pltpu
from jax.experimental.pallas import tpu_sc as plsc

N = 10000
E = 320000
L = 128
STEPS = 10

E1 = 163840
E2 = E - E1

GC = 80
GRB = 5
SNB = 2
SIB = 32
NPAD = 10240
NPT = NPAD // 16

BN = 1000
BE = 2560


def _ln(x):
    m = jnp.mean(x, axis=-1, keepdims=True)
    xc = x - m
    v = jnp.mean(xc * xc, axis=-1, keepdims=True)
    return xc * lax.rsqrt(v + 1e-6)


def _dot(a, b):
    return jnp.dot(a, b, preferred_element_type=jnp.float32)


def _node_enc_body(x, w1, b1, w2, b2, ws, wr, n_out, p_out):
    h = jax.nn.relu(_dot(x[...], w1[...]) + b1[...])
    n0 = _ln(_dot(h, w2[...]) + b2[...])
    n_out[...] = n0
    p_out[0] = _dot(n0, ws[...])
    p_out[1] = _dot(n0, wr[...])


def _node_enc(x, w1, b1, w2, b2, ws, wr):
    wspec = pl.BlockSpec((L, L), lambda i: (0, 0))
    bspec = pl.BlockSpec((1, L), lambda i: (0, 0))
    n0, p = pl.pallas_call(
        _node_enc_body,
        grid=(N // BN,),
        in_specs=[pl.BlockSpec((BN, L), lambda i: (i, 0)),
                  wspec, bspec, wspec, bspec, wspec, wspec],
        out_specs=[pl.BlockSpec((BN, L), lambda i: (i, 0)),
                   pl.BlockSpec((2, BN, L), lambda i: (0, i, 0))],
        out_shape=[jax.ShapeDtypeStruct((N, L), jnp.float32),
                   jax.ShapeDtypeStruct((2, N, L), jnp.float32)],
    )(x, w1, b1, w2, b2, ws, wr)
    return n0, p.reshape(2 * N, L)


def _edge_enc_body(x, w1, b1, w2, b2, e_out):
    h = jax.nn.relu(_dot(x[...], w1[...]) + b1[...])
    e_out[...] = _ln(_dot(h, w2[...]) + b2[...])


def _edge_enc(x, w1, b1, w2, b2):
    ne, d_in = x.shape
    return pl.pallas_call(
        _edge_enc_body,
        grid=(ne // BE,),
        in_specs=[pl.BlockSpec((BE, d_in), lambda i: (i, 0)),
                  pl.BlockSpec((d_in, L), lambda i: (0, 0)),
                  pl.BlockSpec((1, L), lambda i: (0, 0)),
                  pl.BlockSpec((L, L), lambda i: (0, 0)),
                  pl.BlockSpec((1, L), lambda i: (0, 0))],
        out_specs=pl.BlockSpec((BE, L), lambda i: (i, 0)),
        out_shape=jax.ShapeDtypeStruct((ne, L), jnp.float32),
    )(x, w1, b1, w2, b2)


def _edge_step_body(e, g, w1e, b1, w2, b2, ep_out, eo_out):
    h = jax.nn.relu(_dot(e[...], w1e[...]) + g[...] + b1[...])
    en = _ln(_dot(h, w2[...]) + b2[...])
    eo_out[...] = e[...] + en
    y = en.astype(jnp.bfloat16).reshape(BE // 2, 2, L)
    u0 = lax.bitcast_convert_type(y[:, 0, :], jnp.uint16).astype(jnp.uint32)
    u1 = lax.bitcast_convert_type(y[:, 1, :], jnp.uint16).astype(jnp.uint32)
    ep_out[...] = lax.bitcast_convert_type(u0 | (u1 << 16), jnp.int32)


def _edge_step(e, g, w1e, b1, w2, b2):
    ne = e.shape[0]
    wspec = pl.BlockSpec((L, L), lambda i: (0, 0))
    bspec = pl.BlockSpec((1, L), lambda i: (0, 0))
    return pl.pallas_call(
        _edge_step_body,
        grid=(ne // BE,),
        in_specs=[pl.BlockSpec((BE, L), lambda i: (i, 0)),
                  pl.BlockSpec((BE, L), lambda i: (i, 0)),
                  wspec, bspec, wspec, bspec],
        out_specs=[pl.BlockSpec((BE // 2, L), lambda i: (i, 0)),
                   pl.BlockSpec((BE, L), lambda i: (i, 0))],
        out_shape=[jax.ShapeDtypeStruct((ne // 2, L), jnp.int32),
                   jax.ShapeDtypeStruct((ne, L), jnp.float32)],
    )(e, g, w1e, b1, w2, b2)


def _node_step_body(n, a1s, a1r, a2s, a2r, v1n, v1s, v1r, c1, v2, c2,
                    ws, wr, n_out, p_out):
    ags = a1s[0] + a2s[0]
    agr = a1r[0] + a2r[0]
    h = jax.nn.relu(_dot(n[...], v1n[...]) + _dot(ags, v1s[...])
                    + _dot(agr, v1r[...]) + c1[...])
    nn = _ln(_dot(h, v2[...]) + c2[...])
    n1 = n[...] + nn
    n_out[...] = n1
    p_out[0] = _dot(n1, ws[...])
    p_out[1] = _dot(n1, wr[...])


def _node_step(n, agg1, agg2, v1n, v1s, v1r, c1, v2, c2, ws, wr):
    wspec = pl.BlockSpec((L, L), lambda i: (0, 0))
    bspec = pl.BlockSpec((1, L), lambda i: (0, 0))
    aspec_s = pl.BlockSpec((1, BN, L), lambda i: (0, i, 0))
    aspec_r = pl.BlockSpec((1, BN, L), lambda i: (1, i, 0))
    n1, p = pl.pallas_call(
        _node_step_body,
        grid=(N // BN,),
        in_specs=[pl.BlockSpec((BN, L), lambda i: (i, 0)),
                  aspec_s, aspec_r, aspec_s, aspec_r,
                  wspec, wspec, wspec, bspec, wspec, bspec, wspec, wspec],
        out_specs=[pl.BlockSpec((BN, L), lambda i: (i, 0)),
                   pl.BlockSpec((2, BN, L), lambda i: (0, i, 0))],
        out_shape=[jax.ShapeDtypeStruct((N, L), jnp.float32),
                   jax.ShapeDtypeStruct((2, N, L), jnp.float32)],
    )(n, agg1, agg1, agg2, agg2,
      v1n, v1s, v1r, c1, v2, c2, ws, wr)
    return n1, p.reshape(2 * N, L)


def _dec_body(n, w1, b1, w2, b2, out):
    h = jax.nn.relu(_dot(n[...], w1[...]) + b1[...])
    out[...] = _dot(h, w2[...]) + b2[...]


def _dec(n, w1, b1, w2, b2):
    return pl.pallas_call(
        _dec_body,
        grid=(N // BN,),
        in_specs=[pl.BlockSpec((BN, L), lambda i: (i, 0)),
                  pl.BlockSpec((L, L), lambda i: (0, 0)),
                  pl.BlockSpec((1, L), lambda i: (0, 0)),
                  pl.BlockSpec((L, L), lambda i: (0, 0)),
                  pl.BlockSpec((1, L), lambda i: (0, 0))],
        out_specs=pl.BlockSpec((BN, L), lambda i: (i, 0)),
        out_shape=jax.ShapeDtypeStruct((N, L), jnp.float32),
    )(n, w1, b1, w2, b2)


def _make_gather(ne):
    ept = ne // 32
    cnt = ept // GC
    assert cnt > GRB
    mesh = plsc.VectorSubcoreMesh(core_axis_name="c", subcore_axis_name="s")

    @functools.partial(
        pl.kernel,
        out_type=jax.ShapeDtypeStruct((ne, L), jnp.float32),
        mesh=mesh,
        scratch_types=[
            pltpu.VMEM((2, cnt, GC), jnp.int32),
            pltpu.VMEM((GRB, 2, GC, L), jnp.float32),
            pltpu.SemaphoreType.DMA((GRB,)),
            pltpu.SemaphoreType.DMA((GRB,)),
            pltpu.SemaphoreType.DMA((GRB,)),
        ],
    )
    def k(table_hbm, idx_hbm, out_hbm, idx_v, bufs, asem, bsem, ssem):
        c = lax.axis_index("c")
        s = lax.axis_index("s")
        base0 = (s * 2 + c) * ept
        pltpu.sync_copy(idx_hbm.at[s, c], idx_v)

        def gath(j, b, half, sem):
            return pltpu.make_async_copy(
                table_hbm.at[idx_v.at[half, j]], bufs.at[b, half], sem.at[b])

        def stor(j, b):
            dst = out_hbm.at[pl.ds(pl.multiple_of(base0 + j * GC, GC), GC)]
            return pltpu.make_async_copy(bufs.at[b, 0], dst, ssem.at[b])

        def add_pair(b):
            def row(r, carry):
                for rr in range(2):
                    for u in range(L // 16):
                        ri, cs = 2 * r + rr, pl.ds(16 * u, 16)
                        bufs[b, 0, ri, cs] = (bufs[b, 0, ri, cs]
                                              + bufs[b, 1, ri, cs])
                return carry

            lax.fori_loop(0, GC // 2, row, 0)

        def fire(j, b):
            gath(j, b, 0, asem).start()
            gath(j, b, 1, bsem).start()

        def consume(j, b):
            gath(j, b, 0, asem).wait()
            gath(j, b, 1, bsem).wait()
            add_pair(b)
            stor(j, b).start()

        for b in range(GRB - 1):
            fire(b, b)

        def outer(i, carry):
            for b in range(GRB):
                j = i * GRB + b
                consume(j, b)
                bn = (b + GRB - 1) % GRB
                jn = j + GRB - 1

                @pl.when(jn < cnt)
                def _():
                    @pl.when(j >= 1)
                    def _():
                        stor(j - 1, bn).wait()
                    fire(jn, bn)

            return carry

        lax.fori_loop(0, cnt // GRB, outer, 0)
        for t in range(cnt % GRB):
            j = (cnt // GRB) * GRB + t
            consume(j, j % GRB)
        for t in range(GRB):
            j = cnt - GRB + t
            stor(j, j % GRB).wait()

    return k


def _make_scatter(ne):
    srt = ne // 16
    cnt = srt // GC
    hc = GC // 2
    blocks = []
    bc = 0
    while bc < cnt:
        blocks.append((bc, min(SIB, cnt - bc)))
        bc += SIB
    mesh = plsc.VectorSubcoreMesh(core_axis_name="c", subcore_axis_name="s")

    @functools.partial(
        pl.kernel,
        out_type=jax.ShapeDtypeStruct((2, NPAD, L), jnp.float32),
        mesh=mesh,
        scratch_types=[
            pltpu.VMEM((SIB, GC), jnp.int32),
            pltpu.VMEM((SNB, hc, L), jnp.int32),
            pltpu.VMEM((SNB, GC, L), jnp.float32),
            pltpu.VMEM_SHARED((NPAD, L), jnp.float32),
            pltpu.SemaphoreType.DMA((SNB,)),
            pltpu.SemaphoreType.DMA((SNB,)),
        ],
    )
    def k(enp_hbm, idx_hbm, zeros_hbm, out_hbm, idx_v, pbuf, rows, acc,
          lsem, asem):
        c = lax.axis_index("c")
        s = lax.axis_index("s")
        row0 = s * NPT

        pltpu.sync_copy(zeros_hbm, acc.at[pl.ds(row0, NPT)])
        plsc.subcore_barrier()

        base0 = s * (srt // 2)
        msk = jnp.int32(-65536)

        def run_block(blk, bcnt):
            def load(j, b):
                off = pl.multiple_of(base0 + (blk + j) * hc, hc)
                return pltpu.make_async_copy(
                    enp_hbm.at[pl.ds(off, hc)], pbuf.at[b], lsem.at[b])

            def scat(j, b):
                return pltpu.make_async_copy(
                    rows.at[b], acc.at[idx_v.at[j]], asem.at[b])

            def unpack(b):
                def row(r, carry):
                    for u in range(L // 16):
                        cs = pl.ds(16 * u, 16)
                        x = pbuf[b, r, cs]
                        rows[b, 2 * r, cs] = lax.bitcast_convert_type(
                            jnp.left_shift(x, 16), jnp.float32)
                        rows[b, 2 * r + 1, cs] = lax.bitcast_convert_type(
                            jnp.bitwise_and(x, msk), jnp.float32)
                    return carry

                lax.fori_loop(0, hc, row, 0)

            for b in range(SNB - 1):
                load(b, b).start()

            def outer(i, carry):
                for b in range(SNB):
                    j = i * SNB + b
                    load(j, b).wait()
                    bn = (b + SNB - 1) % SNB
                    jn = j + SNB - 1

                    @pl.when(jn < bcnt)
                    def _():
                        load(jn, bn).start()

                    @pl.when(j >= SNB)
                    def _():
                        scat(j - SNB, b).wait()
                    unpack(b)
                    scat(j, b).start(add=True)

                return carry

            lax.fori_loop(0, bcnt // SNB, outer, 0)
            for t in range(bcnt % SNB):
                j = (bcnt // SNB) * SNB + t
                b = j % SNB
                load(j, b).wait()

                @pl.when(j >= SNB)
                def _():
                    scat(j - SNB, b).wait()
                unpack(b)
                scat(j, b).start(add=True)
            for t in range(SNB):
                j = bcnt - SNB + t
                scat(j, j % SNB).wait()

        for blk, bcnt in blocks:
            pltpu.sync_copy(idx_hbm.at[c, s, pl.ds(blk, SIB)], idx_v)
            run_block(blk, bcnt)
        plsc.subcore_barrier()

        pltpu.sync_copy(acc.at[pl.ds(row0, NPT)],
                        out_hbm.at[c, pl.ds(row0, NPT)])

    return k


_gather1 = _make_gather(E1)
_gather2 = _make_gather(E2)
_scatter1 = _make_scatter(E1)
_scatter2 = _make_scatter(E2)


def _gplan(sh, rh):
    cnt = sh.shape[0] // 32 // GC
    return jnp.stack([sh.reshape(16, 2, cnt, GC),
                      (rh + N).reshape(16, 2, cnt, GC)], axis=2)


def _splan(sh, rh):
    cnt = sh.shape[0] // 16 // GC
    x = jnp.stack([sh, rh]).reshape(2, 16, cnt, GC)
    pad = (-cnt) % SIB
    return jnp.pad(x, ((0, 0), (0, 0), (0, pad), (0, 0)))


def kernel(nodes, edges, senders, receivers, params):
    (we1, be1), (we2, be2) = params['enc_node']
    (wf1, bf1), (wf2, bf2) = params['enc_edge']
    (w1, b1), (w2, b2) = params['upd_edge']
    (v1, c1), (v2, c2) = params['upd_node']
    (wd1, bd1), (wd2, bd2) = params['dec_node']

    w1e, w1s, w1r = w1[:L], w1[L:2 * L], w1[2 * L:]
    v1n, v1s, v1r = v1[:L], v1[L:2 * L], v1[2 * L:]
    b1r, b2r = b1.reshape(1, L), b2.reshape(1, L)
    c1r, c2r = c1.reshape(1, L), c2.reshape(1, L)

    s32 = senders.astype(jnp.int32)
    r32 = receivers.astype(jnp.int32)
    g1p = _gplan(s32[:E1], r32[:E1])
    g2p = _gplan(s32[E1:], r32[E1:])
    s1p = _splan(s32[:E1], r32[:E1])
    s2p = _splan(s32[E1:], r32[E1:])
    zeros = jnp.zeros((NPT, L), jnp.float32)

    n, p = _node_enc(nodes, we1, be1.reshape(1, L), we2, be2.reshape(1, L),
                     w1s, w1r)
    bf1r, bf2r = bf1.reshape(1, L), bf2.reshape(1, L)
    e1 = _edge_enc(edges[:E1], wf1, bf1r, wf2, bf2r)
    e2 = _edge_enc(edges[E1:], wf1, bf1r, wf2, bf2r)

    for _ in range(STEPS):
        g1 = _gather1(p, g1p)
        g2 = _gather2(p, g2p)
        ep1, e1 = _edge_step(e1, g1, w1e, b1r, w2, b2r)
        ep2, e2 = _edge_step(e2, g2, w1e, b1r, w2, b2r)
        agg1 = _scatter1(ep1, s1p, zeros)
        agg2 = _scatter2(ep2, s2p, zeros)
        n, p = _node_step(n, agg1, agg2, v1n, v1s, v1r, c1r, v2, c2r,
                          w1s, w1r)

    wd2p = jnp.zeros((L, L), jnp.float32).at[:, :wd2.shape[1]].set(wd2)
    bd2p = jnp.zeros((1, L), jnp.float32).at[0, :bd2.shape[0]].set(bd2)
    out = _dec(n, wd1, bd1.reshape(1, L), wd2p, bd2p)
    return out[:, :wd2.shape[1]]


# speedup vs baseline: 1.2808x; 1.2808x over previous
import functools

import jax
import jax.numpy as jnp
from jax import lax
from jax.experimental import pallas as pl
from jax.experimental.pallas import tpu as pltpu
from jax.experimental.pallas import tpu_sc as plsc

N = 10000
E = 320000
L = 128
STEPS = 10

E1 = 163840
E2 = E - E1

GC = 80
GRB = 5
SNB = 3
SIB = 64
NPAD = 10240
NPT = NPAD // 16

BN = 1000
BE = 2560


def _ln(x):
    m = jnp.mean(x, axis=-1, keepdims=True)
    xc = x - m
    v = jnp.mean(xc * xc, axis=-1, keepdims=True)
    return xc * lax.rsqrt(v + 1e-6)


def _dot(a, b):
    return jnp.dot(a, b, preferred_element_type=jnp.float32)


def _node_enc_body(x, w1, b1, w2, b2, ws, wr, n_out, p_out):
    h = jax.nn.relu(_dot(x[...], w1[...]) + b1[...])
    n0 = _ln(_dot(h, w2[...]) + b2[...])
    n_out[...] = n0
    p_out[0] = _dot(n0, ws[...])
    p_out[1] = _dot(n0, wr[...])


def _node_enc(x, w1, b1, w2, b2, ws, wr):
    wspec = pl.BlockSpec((L, L), lambda i: (0, 0))
    bspec = pl.BlockSpec((1, L), lambda i: (0, 0))
    n0, p = pl.pallas_call(
        _node_enc_body,
        grid=(N // BN,),
        in_specs=[pl.BlockSpec((BN, L), lambda i: (i, 0)),
                  wspec, bspec, wspec, bspec, wspec, wspec],
        out_specs=[pl.BlockSpec((BN, L), lambda i: (i, 0)),
                   pl.BlockSpec((2, BN, L), lambda i: (0, i, 0))],
        out_shape=[jax.ShapeDtypeStruct((N, L), jnp.float32),
                   jax.ShapeDtypeStruct((2, N, L), jnp.float32)],
    )(x, w1, b1, w2, b2, ws, wr)
    return n0, p.reshape(2 * N, L)


def _edge_enc_body(x, w1, b1, w2, b2, e_out):
    h = jax.nn.relu(_dot(x[...], w1[...]) + b1[...])
    e_out[...] = _ln(_dot(h, w2[...]) + b2[...])


def _edge_enc(x, w1, b1, w2, b2):
    ne, d_in = x.shape
    return pl.pallas_call(
        _edge_enc_body,
        grid=(ne // BE,),
        in_specs=[pl.BlockSpec((BE, d_in), lambda i: (i, 0)),
                  pl.BlockSpec((d_in, L), lambda i: (0, 0)),
                  pl.BlockSpec((1, L), lambda i: (0, 0)),
                  pl.BlockSpec((L, L), lambda i: (0, 0)),
                  pl.BlockSpec((1, L), lambda i: (0, 0))],
        out_specs=pl.BlockSpec((BE, L), lambda i: (i, 0)),
        out_shape=jax.ShapeDtypeStruct((ne, L), jnp.float32),
    )(x, w1, b1, w2, b2)


def _edge_step_body(e, g, w1e, b1, w2, b2, en_out, eo_out):
    h = jax.nn.relu(_dot(e[...], w1e[...]) + g[...] + b1[...])
    en = _ln(_dot(h, w2[...]) + b2[...])
    en_out[...] = en
    eo_out[...] = e[...] + en


def _edge_step(e, g, w1e, b1, w2, b2):
    ne = e.shape[0]
    wspec = pl.BlockSpec((L, L), lambda i: (0, 0))
    bspec = pl.BlockSpec((1, L), lambda i: (0, 0))
    return pl.pallas_call(
        _edge_step_body,
        grid=(ne // BE,),
        in_specs=[pl.BlockSpec((BE, L), lambda i: (i, 0)),
                  pl.BlockSpec((BE, L), lambda i: (i, 0)),
                  wspec, bspec, wspec, bspec],
        out_specs=[pl.BlockSpec((BE, L), lambda i: (i, 0)),
                   pl.BlockSpec((BE, L), lambda i: (i, 0))],
        out_shape=[jax.ShapeDtypeStruct((ne, L), jnp.float32),
                   jax.ShapeDtypeStruct((ne, L), jnp.float32)],
    )(e, g, w1e, b1, w2, b2)


def _node_step_body(n, a1s, a1r, a2s, a2r, v1n, v1s, v1r, c1, v2, c2,
                    ws, wr, n_out, p_out):
    ags = a1s[0] + a2s[0]
    agr = a1r[0] + a2r[0]
    h = jax.nn.relu(_dot(n[...], v1n[...]) + _dot(ags, v1s[...])
                    + _dot(agr, v1r[...]) + c1[...])
    nn = _ln(_dot(h, v2[...]) + c2[...])
    n1 = n[...] + nn
    n_out[...] = n1
    p_out[0] = _dot(n1, ws[...])
    p_out[1] = _dot(n1, wr[...])


def _node_step(n, agg1, agg2, v1n, v1s, v1r, c1, v2, c2, ws, wr):
    wspec = pl.BlockSpec((L, L), lambda i: (0, 0))
    bspec = pl.BlockSpec((1, L), lambda i: (0, 0))
    aspec_s = pl.BlockSpec((1, BN, L), lambda i: (0, i, 0))
    aspec_r = pl.BlockSpec((1, BN, L), lambda i: (1, i, 0))
    n1, p = pl.pallas_call(
        _node_step_body,
        grid=(N // BN,),
        in_specs=[pl.BlockSpec((BN, L), lambda i: (i, 0)),
                  aspec_s, aspec_r, aspec_s, aspec_r,
                  wspec, wspec, wspec, bspec, wspec, bspec, wspec, wspec],
        out_specs=[pl.BlockSpec((BN, L), lambda i: (i, 0)),
                   pl.BlockSpec((2, BN, L), lambda i: (0, i, 0))],
        out_shape=[jax.ShapeDtypeStruct((N, L), jnp.float32),
                   jax.ShapeDtypeStruct((2, N, L), jnp.float32)],
    )(n, agg1, agg1, agg2, agg2,
      v1n, v1s, v1r, c1, v2, c2, ws, wr)
    return n1, p.reshape(2 * N, L)


def _dec_body(n, w1, b1, w2, b2, out):
    h = jax.nn.relu(_dot(n[...], w1[...]) + b1[...])
    out[...] = _dot(h, w2[...]) + b2[...]


def _dec(n, w1, b1, w2, b2):
    return pl.pallas_call(
        _dec_body,
        grid=(N // BN,),
        in_specs=[pl.BlockSpec((BN, L), lambda i: (i, 0)),
                  pl.BlockSpec((L, L), lambda i: (0, 0)),
                  pl.BlockSpec((1, L), lambda i: (0, 0)),
                  pl.BlockSpec((L, L), lambda i: (0, 0)),
                  pl.BlockSpec((1, L), lambda i: (0, 0))],
        out_specs=pl.BlockSpec((BN, L), lambda i: (i, 0)),
        out_shape=jax.ShapeDtypeStruct((N, L), jnp.float32),
    )(n, w1, b1, w2, b2)


def _make_gather(ne):
    ept = ne // 32
    cnt = ept // GC
    assert cnt > GRB
    mesh = plsc.VectorSubcoreMesh(core_axis_name="c", subcore_axis_name="s")

    @functools.partial(
        pl.kernel,
        out_type=jax.ShapeDtypeStruct((ne, L), jnp.float32),
        mesh=mesh,
        scratch_types=[
            pltpu.VMEM((2, cnt, GC), jnp.int32),
            pltpu.VMEM((GRB, 2, GC, L), jnp.float32),
            pltpu.SemaphoreType.DMA((GRB,)),
            pltpu.SemaphoreType.DMA((GRB,)),
            pltpu.SemaphoreType.DMA((GRB,)),
        ],
    )
    def k(table_hbm, idx_hbm, out_hbm, idx_v, bufs, asem, bsem, ssem):
        c = lax.axis_index("c")
        s = lax.axis_index("s")
        base0 = (s * 2 + c) * ept
        pltpu.sync_copy(idx_hbm.at[s, c], idx_v)

        def gath(j, b, half, sem):
            return pltpu.make_async_copy(
                table_hbm.at[idx_v.at[half, j]], bufs.at[b, half], sem.at[b])

        def stor(j, b):
            dst = out_hbm.at[pl.ds(pl.multiple_of(base0 + j * GC, GC), GC)]
            return pltpu.make_async_copy(bufs.at[b, 0], dst, ssem.at[b])

        def add_pair(b):
            def row(r, carry):
                for rr in range(2):
                    for u in range(L // 16):
                        ri, cs = 2 * r + rr, pl.ds(16 * u, 16)
                        bufs[b, 0, ri, cs] = (bufs[b, 0, ri, cs]
                                              + bufs[b, 1, ri, cs])
                return carry

            lax.fori_loop(0, GC // 2, row, 0)

        def fire(j, b):
            gath(j, b, 0, asem).start()
            gath(j, b, 1, bsem).start()

        def consume(j, b):
            gath(j, b, 0, asem).wait()
            gath(j, b, 1, bsem).wait()
            add_pair(b)
            stor(j, b).start()

        for b in range(GRB - 1):
            fire(b, b)

        def outer(i, carry):
            for b in range(GRB):
                j = i * GRB + b
                consume(j, b)
                bn = (b + GRB - 1) % GRB
                jn = j + GRB - 1

                @pl.when(jn < cnt)
                def _():
                    @pl.when(j >= 1)
                    def _():
                        stor(j - 1, bn).wait()
                    fire(jn, bn)

            return carry

        lax.fori_loop(0, cnt // GRB, outer, 0)
        for t in range(cnt % GRB):
            j = (cnt // GRB) * GRB + t
            consume(j, j % GRB)
        for t in range(GRB):
            j = cnt - GRB + t
            stor(j, j % GRB).wait()

    return k


def _make_scatter(ne):
    srt = ne // 16
    cnt = srt // GC
    blocks = []
    bc = 0
    while bc < cnt:
        blocks.append((bc, min(SIB, cnt - bc)))
        bc += SIB
    mesh = plsc.VectorSubcoreMesh(core_axis_name="c", subcore_axis_name="s")

    @functools.partial(
        pl.kernel,
        out_type=jax.ShapeDtypeStruct((2, NPAD, L), jnp.float32),
        mesh=mesh,
        scratch_types=[
            pltpu.VMEM((SIB, GC), jnp.int32),
            pltpu.VMEM((SNB, GC, L), jnp.float32),
            pltpu.VMEM_SHARED((NPAD, L), jnp.float32),
            pltpu.SemaphoreType.DMA((SNB,)),
            pltpu.SemaphoreType.DMA((SNB,)),
        ],
    )
    def k(enew_hbm, idx_hbm, zeros_hbm, out_hbm, idx_v, rows, acc,
          lsem, asem):
        c = lax.axis_index("c")
        s = lax.axis_index("s")
        row0 = s * NPT

        pltpu.sync_copy(zeros_hbm, acc.at[pl.ds(row0, NPT)])
        plsc.subcore_barrier()

        base0 = s * srt

        def run_block(blk, bcnt):
            def load(j, b):
                off = pl.multiple_of(base0 + (blk + j) * GC, GC)
                return pltpu.make_async_copy(
                    enew_hbm.at[pl.ds(off, GC)], rows.at[b], lsem.at[b])

            def scat(j, b):
                return pltpu.make_async_copy(
                    rows.at[b], acc.at[idx_v.at[j]], asem.at[b])

            for b in range(SNB - 1):
                load(b, b).start()

            def outer(i, carry):
                for b in range(SNB):
                    j = i * SNB + b
                    load(j, b).wait()
                    bn = (b + SNB - 1) % SNB
                    jn = j + SNB - 1

                    @pl.when(j >= 1)
                    def _():
                        scat(j - 1, bn).wait()
                    scat(j, b).start(add=True)

                    @pl.when(jn < bcnt)
                    def _():
                        load(jn, bn).start()

                return carry

            lax.fori_loop(0, bcnt // SNB, outer, 0)
            for t in range(bcnt % SNB):
                j = (bcnt // SNB) * SNB + t
                b = j % SNB
                load(j, b).wait()
                scat(j - 1, (b + SNB - 1) % SNB).wait()
                scat(j, b).start(add=True)
            scat(bcnt - 1, (bcnt - 1) % SNB).wait()

        for blk, bcnt in blocks:
            pltpu.sync_copy(idx_hbm.at[c, s, pl.ds(blk, SIB)], idx_v)
            run_block(blk, bcnt)
        plsc.subcore_barrier()

        pltpu.sync_copy(acc.at[pl.ds(row0, NPT)],
                        out_hbm.at[c, pl.ds(row0, NPT)])

    return k


_gather1 = _make_gather(E1)
_gather2 = _make_gather(E2)
_scatter1 = _make_scatter(E1)
_scatter2 = _make_scatter(E2)


def _gplan(sh, rh):
    cnt = sh.shape[0] // 32 // GC
    return jnp.stack([sh.reshape(16, 2, cnt, GC),
                      (rh + N).reshape(16, 2, cnt, GC)], axis=2)


def _splan(sh, rh):
    cnt = sh.shape[0] // 16 // GC
    x = jnp.stack([sh, rh]).reshape(2, 16, cnt, GC)
    pad = (-cnt) % SIB
    return jnp.pad(x, ((0, 0), (0, 0), (0, pad), (0, 0)))


def kernel(nodes, edges, senders, receivers, params):
    (we1, be1), (we2, be2) = params['enc_node']
    (wf1, bf1), (wf2, bf2) = params['enc_edge']
    (w1, b1), (w2, b2) = params['upd_edge']
    (v1, c1), (v2, c2) = params['upd_node']
    (wd1, bd1), (wd2, bd2) = params['dec_node']

    w1e, w1s, w1r = w1[:L], w1[L:2 * L], w1[2 * L:]
    v1n, v1s, v1r = v1[:L], v1[L:2 * L], v1[2 * L:]
    b1r, b2r = b1.reshape(1, L), b2.reshape(1, L)
    c1r, c2r = c1.reshape(1, L), c2.reshape(1, L)

    s32 = senders.astype(jnp.int32)
    r32 = receivers.astype(jnp.int32)
    g1p = _gplan(s32[:E1], r32[:E1])
    g2p = _gplan(s32[E1:], r32[E1:])
    s1p = _splan(s32[:E1], r32[:E1])
    s2p = _splan(s32[E1:], r32[E1:])
    zeros = jnp.zeros((NPT, L), jnp.float32)

    n, p = _node_enc(nodes, we1, be1.reshape(1, L), we2, be2.reshape(1, L),
                     w1s, w1r)
    bf1r, bf2r = bf1.reshape(1, L), bf2.reshape(1, L)
    e1 = _edge_enc(edges[:E1], wf1, bf1r, wf2, bf2r)
    e2 = _edge_enc(edges[E1:], wf1, bf1r, wf2, bf2r)

    for _ in range(STEPS):
        g1 = _gather1(p, g1p)
        g2 = _gather2(p, g2p)
        en1, e1 = _edge_step(e1, g1, w1e, b1r, w2, b2r)
        en2, e2 = _edge_step(e2, g2, w1e, b1r, w2, b2r)
        agg1 = _scatter1(en1, s1p, zeros)
        agg2 = _scatter2(en2, s2p, zeros)
        n, p = _node_step(n, agg1, agg2, v1n, v1s, v1r, c1r, v2, c2r,
                          w1s, w1r)

    wd2p = jnp.zeros((L, L), jnp.float32).at[:, :wd2.shape[1]].set(wd2)
    bd2p = jnp.zeros((1, L), jnp.float32).at[0, :bd2.shape[0]].set(bd2)
    out = _dec(n, wd1, bd1.reshape(1, L), wd2p, bd2p)
    return out[:, :wd2.shape[1]]
